# baseline (speedup 1.0000x reference)
import functools

import jax
import jax.numpy as jnp
from jax import lax
from jax.experimental import pallas as pl
from jax.experimental.pallas import tpu as pltpu
from jax.experimental.pallas import tpu_sc as plsc

D = 64
NC = 2
NS = 16
NW = NC * NS
C = 128
G = 16
EPS = 1e-12

_SHUF_DN = lax.GatherDimensionNumbers(
    offset_dims=(), collapsed_slice_dims=(0,), start_index_map=(0,))


def _lane_shuffle(x, p):
    return lax.gather(x, p[:, None], _SHUF_DN, (1,),
                      mode=lax.GatherScatterMode.PROMISE_IN_BOUNDS)


def _rsqrt(v):
    i = plsc.bitcast(v, jnp.int32)
    i = jnp.int32(0x5F3759DF) - lax.shift_right_logical(i, 1)
    y = plsc.bitcast(i, jnp.float32)
    half = v * 0.5
    for _ in range(3):
        y = y * (1.5 - half * y * y)
    return y


def _body(seq, nch, ids, tok2, pos, gam, bet, out,
          pos_v, g_v, b_v, idx_b, gidx_b, hb_b, row_b, out_b, s_v, q_v,
          gsem, isem, wsem):
    cid = lax.axis_index("c")
    sid = lax.axis_index("s")
    wid = sid * NC + cid
    per_w = nch * C

    pltpu.sync_copy(pos.at[pl.ds(0, seq)], pos_v)
    pltpu.sync_copy(gam, g_v)
    pltpu.sync_copy(bet, b_v)

    gs = [g_v[pl.ds(16 * k, 16)] for k in range(4)]
    bs = [b_v[pl.ds(16 * k, 16)] for k in range(4)]
    iota = lax.iota(jnp.int32, 16)
    iota17 = iota * 17

    base_w = wid * per_w

    def make_gidx(b):
        for j in range(C // 16):
            v = idx_b[b][pl.ds(16 * j, 16)]
            gidx_b[b][pl.ds(16 * j, 16)] = lax.shift_right_logical(v, 1)
            hb_b[b][pl.ds(16 * j, 16)] = v & 1

    def compute(cur, b):
        row = row_b[b]
        o = out_b[b]
        hb = hb_b[b]
        sb = lax.rem(base_w + cur * C, seq)

        def group(t, _):
            tb = t * G
            idxv = hb[pl.ds(tb, 16)]
            for r in range(G):
                tr = tb + r
                sel = jnp.full((16,), r, jnp.int32)
                hmask = _lane_shuffle(idxv, sel) > 0
                sr = sb + tr
                sr = jnp.where(sr >= seq, sr - seq, sr)
                xs = [jnp.where(hmask,
                                row[tr, pl.ds(D + 16 * k, 16)],
                                row[tr, pl.ds(16 * k, 16)])
                      + pos_v[sr, pl.ds(16 * k, 16)] for k in range(4)]
                for k in range(4):
                    o[tr, pl.ds(16 * k, 16)] = xs[k]
                sv = (xs[0] + xs[1]) + (xs[2] + xs[3])
                qv = (xs[0] * xs[0] + xs[1] * xs[1]) + \
                     (xs[2] * xs[2] + xs[3] * xs[3])
                plsc.store_scatter(s_v, [iota + r * 17], sv)
                plsc.store_scatter(q_v, [iota + r * 17], qv)
            sacc = jnp.zeros((16,), jnp.float32)
            qacc = jnp.zeros((16,), jnp.float32)
            for l in range(16):
                fl = iota17 + l
                sacc = sacc + plsc.load_gather(s_v, [fl])
                qacc = qacc + plsc.load_gather(q_v, [fl])
            mean = sacc * (1.0 / D)
            var = qacc * (1.0 / D) - mean * mean
            rstd = _rsqrt(var + EPS)
            for r in range(G):
                tr = tb + r
                sel = jnp.full((16,), r, jnp.int32)
                mv = _lane_shuffle(mean, sel)
                rv = _lane_shuffle(rstd, sel)
                for k in range(4):
                    x = o[tr, pl.ds(16 * k, 16)]
                    o[tr, pl.ds(16 * k, 16)] = (x - mv) * rv * gs[k] + bs[k]
            return 0

        lax.fori_loop(0, C // G, group, 0)

    def ids_at(c):
        return ids.at[pl.ds(base_w + c * C, C)]

    def out_at(c):
        return out.at[pl.ds(base_w + c * C, C)]

    def gather(b):
        return pltpu.async_copy(tok2.at[gidx_b[b]], row_b[b], gsem[b])

    pltpu.sync_copy(ids_at(0), idx_b[0])
    make_gidx(0)
    gather(0)
    pltpu.async_copy(ids_at(1), idx_b[1], isem[1])

    def step(i, _):
        g2 = i * 2
        for b in (0, 1):
            cur = g2 + b
            ob = 1 - b
            nxt = cur + 1
            @pl.when(nxt < nch)
            def _():
                pltpu.make_async_copy(ids_at(nxt), idx_b[ob], isem[ob]).wait()
                make_gidx(ob)
                gather(ob)
            pltpu.make_async_copy(tok2.at[gidx_b[b]], row_b[b], gsem[b]).wait()
            @pl.when(nxt + 1 < nch)
            def _():
                pltpu.async_copy(ids_at(nxt + 1), idx_b[b], isem[b])
            @pl.when(cur >= 2)
            def _():
                pltpu.make_async_copy(out_b[b], out_at(0), wsem[b]).wait()
            compute(cur, b)
            pltpu.async_copy(out_b[b], out_at(cur), wsem[b])
        return 0

    lax.fori_loop(0, nch // 2, step, 0)
    pltpu.make_async_copy(out_b[0], out_at(0), wsem[0]).wait()
    pltpu.make_async_copy(out_b[1], out_at(0), wsem[1]).wait()


@functools.lru_cache(maxsize=None)
def _build(nrows, seq, interpret=False):
    assert nrows % (NW * C) == 0 and (nrows // (NW * C)) % 2 == 0
    nch = nrows // (NW * C)
    mesh = plsc.VectorSubcoreMesh(
        core_axis_name="c", subcore_axis_name="s",
        num_cores=NC, num_subcores=NS)

    def body(ids, tok2, pos, gam, bet, out,
             pos_v, g_v, b_v, idx0, idx1, gidx0, gidx1, hb0, hb1,
             row0, row1, out0, out1, s_v, q_v,
             gsem0, gsem1, isem0, isem1, wsem0, wsem1):
        _body(seq, nch, ids, tok2, pos, gam, bet, out,
              pos_v, g_v, b_v, (idx0, idx1), (gidx0, gidx1), (hb0, hb1),
              (row0, row1), (out0, out1),
              s_v, q_v, (gsem0, gsem1), (isem0, isem1), (wsem0, wsem1))

    return pl.kernel(
        body,
        out_type=jax.ShapeDtypeStruct((nrows, D), jnp.float32),
        mesh=mesh,
        interpret=interpret,
        compiler_params=pltpu.CompilerParams(needs_layout_passes=False),
        scratch_types=[
            pltpu.VMEM((seq, D), jnp.float32),
            pltpu.VMEM((D,), jnp.float32),
            pltpu.VMEM((D,), jnp.float32),
            pltpu.VMEM((C,), jnp.int32),
            pltpu.VMEM((C,), jnp.int32),
            pltpu.VMEM((C,), jnp.int32),
            pltpu.VMEM((C,), jnp.int32),
            pltpu.VMEM((C,), jnp.int32),
            pltpu.VMEM((C,), jnp.int32),
            pltpu.VMEM((C, 2 * D), jnp.float32),
            pltpu.VMEM((C, 2 * D), jnp.float32),
            pltpu.VMEM((C, D), jnp.float32),
            pltpu.VMEM((C, D), jnp.float32),
            pltpu.VMEM((G * 17,), jnp.float32),
            pltpu.VMEM((G * 17,), jnp.float32),
            pltpu.SemaphoreType.DMA,
            pltpu.SemaphoreType.DMA,
            pltpu.SemaphoreType.DMA,
            pltpu.SemaphoreType.DMA,
            pltpu.SemaphoreType.DMA,
            pltpu.SemaphoreType.DMA,
        ],
    )


def kernel(input_ids, token_table, pos_table, gamma, beta):
    batch, seq = input_ids.shape
    vocab = token_table.shape[0]
    ids_flat = input_ids.reshape(-1).astype(jnp.int32)
    tok2 = jnp.concatenate(
        [token_table[0::2], token_table[1::2]], axis=1)
    out_flat = _build(batch * seq, seq)(
        ids_flat, tok2, pos_table, gamma, beta)
    return out_flat.reshape(batch, seq, D)


# speedup vs baseline: 6.6771x; 6.6771x over previous
import functools

import jax
import jax.numpy as jnp
from jax import lax
from jax.experimental import pallas as pl
from jax.experimental.pallas import tpu as pltpu
from jax.experimental.pallas import tpu_sc as plsc

D = 64
NC = 2
NS = 16
NW = NC * NS
C = 128
G = 16
EPS = 1e-12

_SHUF_DN = lax.GatherDimensionNumbers(
    offset_dims=(), collapsed_slice_dims=(0,), start_index_map=(0,))


def _lane_shuffle(x, p):
    return lax.gather(x, p[:, None], _SHUF_DN, (1,),
                      mode=lax.GatherScatterMode.PROMISE_IN_BOUNDS)


def _rsqrt(v):
    i = plsc.bitcast(v, jnp.int32)
    i = jnp.int32(0x5F3759DF) - lax.shift_right_logical(i, 1)
    y = plsc.bitcast(i, jnp.float32)
    half = v * 0.5
    for _ in range(3):
        y = y * (1.5 - half * y * y)
    return y


def _body(seq, nch, ids, tok2, pos, gam, bet, out,
          pos_v, g_v, b_v, idx_b, gidx_b, hb_b, row_b, out_b, s_v, q_v,
          gsem, isem, wsem):
    cid = lax.axis_index("c")
    sid = lax.axis_index("s")
    wid = sid * NC + cid
    per_w = nch * C

    pltpu.sync_copy(pos.at[pl.ds(0, seq)], pos_v)
    pltpu.sync_copy(gam, g_v)
    pltpu.sync_copy(bet, b_v)

    gs = [g_v[pl.ds(16 * k, 16)] for k in range(4)]
    bs = [b_v[pl.ds(16 * k, 16)] for k in range(4)]
    iota = lax.iota(jnp.int32, 16)
    iota17 = iota * 17

    base_w = wid * per_w

    def make_gidx(b):
        for j in range(C // 16):
            v = idx_b[b][pl.ds(16 * j, 16)]
            gidx_b[b][pl.ds(16 * j, 16)] = lax.shift_right_logical(v, 1)
            hb_b[b][pl.ds(16 * j, 16)] = v & 1

    def compute(cur, b):
        row = row_b[b]
        o = out_b[b]
        hb = hb_b[b]
        sb = lax.rem(base_w + cur * C, seq)

        def group(t, _):
            tb = t * G
            idxv = hb[pl.ds(tb, 16)] * D
            for r in range(G):
                tr = tb + r
                h = idxv[r]
                sr = sb + tr
                sr = jnp.where(sr >= seq, sr - seq, sr)
                xs = [row[tr, pl.ds(h + 16 * k, 16)]
                      + pos_v[sr, pl.ds(16 * k, 16)] for k in range(4)]
                for k in range(4):
                    o[tr, pl.ds(16 * k, 16)] = xs[k]
                sv = (xs[0] + xs[1]) + (xs[2] + xs[3])
                qv = (xs[0] * xs[0] + xs[1] * xs[1]) + \
                     (xs[2] * xs[2] + xs[3] * xs[3])
                plsc.store_scatter(s_v, [iota + r * 17], sv)
                plsc.store_scatter(q_v, [iota + r * 17], qv)
            sacc = jnp.zeros((16,), jnp.float32)
            qacc = jnp.zeros((16,), jnp.float32)
            for l in range(16):
                fl = iota17 + l
                sacc = sacc + plsc.load_gather(s_v, [fl])
                qacc = qacc + plsc.load_gather(q_v, [fl])
            mean = sacc * (1.0 / D)
            var = qacc * (1.0 / D) - mean * mean
            rstd = _rsqrt(var + EPS)
            for r in range(G):
                tr = tb + r
                sel = jnp.full((16,), r, jnp.int32)
                mv = _lane_shuffle(mean, sel)
                rv = _lane_shuffle(rstd, sel)
                for k in range(4):
                    x = o[tr, pl.ds(16 * k, 16)]
                    o[tr, pl.ds(16 * k, 16)] = (x - mv) * rv * gs[k] + bs[k]
            return 0

        lax.fori_loop(0, C // G, group, 0)

    def ids_at(c):
        return ids.at[pl.ds(base_w + c * C, C)]

    def out_at(c):
        return out.at[pl.ds(base_w + c * C, C)]

    def gather(b):
        return pltpu.async_copy(tok2.at[gidx_b[b]], row_b[b], gsem[b])

    pltpu.sync_copy(ids_at(0), idx_b[0])
    make_gidx(0)
    gather(0)
    pltpu.async_copy(ids_at(1), idx_b[1], isem[1])

    def step(i, _):
        g2 = i * 2
        for b in (0, 1):
            cur = g2 + b
            ob = 1 - b
            nxt = cur + 1
            @pl.when(nxt < nch)
            def _():
                pltpu.make_async_copy(ids_at(nxt), idx_b[ob], isem[ob]).wait()
                make_gidx(ob)
                gather(ob)
            pltpu.make_async_copy(tok2.at[gidx_b[b]], row_b[b], gsem[b]).wait()
            @pl.when(nxt + 1 < nch)
            def _():
                pltpu.async_copy(ids_at(nxt + 1), idx_b[b], isem[b])
            @pl.when(cur >= 2)
            def _():
                pltpu.make_async_copy(out_b[b], out_at(0), wsem[b]).wait()
            compute(cur, b)
            pltpu.async_copy(out_b[b], out_at(cur), wsem[b])
        return 0

    lax.fori_loop(0, nch // 2, step, 0)
    pltpu.make_async_copy(out_b[0], out_at(0), wsem[0]).wait()
    pltpu.make_async_copy(out_b[1], out_at(0), wsem[1]).wait()


@functools.lru_cache(maxsize=None)
def _build(nrows, seq, interpret=False):
    assert nrows % (NW * C) == 0 and (nrows // (NW * C)) % 2 == 0
    nch = nrows // (NW * C)
    mesh = plsc.VectorSubcoreMesh(
        core_axis_name="c", subcore_axis_name="s",
        num_cores=NC, num_subcores=NS)

    def body(ids, tok2, pos, gam, bet, out,
             pos_v, g_v, b_v, idx0, idx1, gidx0, gidx1, hb0, hb1,
             row0, row1, out0, out1, s_v, q_v,
             gsem0, gsem1, isem0, isem1, wsem0, wsem1):
        _body(seq, nch, ids, tok2, pos, gam, bet, out,
              pos_v, g_v, b_v, (idx0, idx1), (gidx0, gidx1), (hb0, hb1),
              (row0, row1), (out0, out1),
              s_v, q_v, (gsem0, gsem1), (isem0, isem1), (wsem0, wsem1))

    return pl.kernel(
        body,
        out_type=jax.ShapeDtypeStruct((nrows, D), jnp.float32),
        mesh=mesh,
        interpret=interpret,
        compiler_params=pltpu.CompilerParams(needs_layout_passes=False),
        scratch_types=[
            pltpu.VMEM((seq, D), jnp.float32),
            pltpu.VMEM((D,), jnp.float32),
            pltpu.VMEM((D,), jnp.float32),
            pltpu.VMEM((C,), jnp.int32),
            pltpu.VMEM((C,), jnp.int32),
            pltpu.VMEM((C,), jnp.int32),
            pltpu.VMEM((C,), jnp.int32),
            pltpu.VMEM((C,), jnp.int32),
            pltpu.VMEM((C,), jnp.int32),
            pltpu.VMEM((C, 2 * D), jnp.float32),
            pltpu.VMEM((C, 2 * D), jnp.float32),
            pltpu.VMEM((C, D), jnp.float32),
            pltpu.VMEM((C, D), jnp.float32),
            pltpu.VMEM((G * 17,), jnp.float32),
            pltpu.VMEM((G * 17,), jnp.float32),
            pltpu.SemaphoreType.DMA,
            pltpu.SemaphoreType.DMA,
            pltpu.SemaphoreType.DMA,
            pltpu.SemaphoreType.DMA,
            pltpu.SemaphoreType.DMA,
            pltpu.SemaphoreType.DMA,
        ],
    )


def kernel(input_ids, token_table, pos_table, gamma, beta):
    batch, seq = input_ids.shape
    vocab = token_table.shape[0]
    ids_flat = input_ids.reshape(-1).astype(jnp.int32)
    tok2 = token_table.reshape(vocab // 2, 2 * D)
    out_flat = _build(batch * seq, seq)(
        ids_flat, tok2, pos_table, gamma, beta)
    return out_flat.reshape(batch, seq, D)


# speedup vs baseline: 7.1514x; 1.0710x over previous
import functools

import jax
import jax.numpy as jnp
from jax import lax
from jax.experimental import pallas as pl
from jax.experimental.pallas import tpu as pltpu
from jax.experimental.pallas import tpu_sc as plsc

D = 64
NC = 2
NS = 16
NW = NC * NS
C = 128
G = 16
EPS = 1e-12

_SHUF_DN = lax.GatherDimensionNumbers(
    offset_dims=(), collapsed_slice_dims=(0,), start_index_map=(0,))


def _lane_shuffle(x, p):
    return lax.gather(x, p[:, None], _SHUF_DN, (1,),
                      mode=lax.GatherScatterMode.PROMISE_IN_BOUNDS)


def _rsqrt(v):
    i = plsc.bitcast(v, jnp.int32)
    i = jnp.int32(0x5F3759DF) - lax.shift_right_logical(i, 1)
    y = plsc.bitcast(i, jnp.float32)
    half = v * 0.5
    for _ in range(3):
        y = y * (1.5 - half * y * y)
    return y


def _body(seq, nch, ids, tok2, pos, gam, bet, out,
          pos_v, g_v, b_v, idx_b, gidx_b, hb_b, row_b, out_b, s_v, q_v,
          gsem, isem, wsem):
    cid = lax.axis_index("c")
    sid = lax.axis_index("s")
    wid = sid * NC + cid
    per_w = nch * C

    pltpu.sync_copy(pos.at[pl.ds(0, seq)], pos_v)
    pltpu.sync_copy(gam, g_v)
    pltpu.sync_copy(bet, b_v)

    gs = [g_v[pl.ds(16 * k, 16)] for k in range(4)]
    bs = [b_v[pl.ds(16 * k, 16)] for k in range(4)]
    iota = lax.iota(jnp.int32, 16)
    iota17 = iota * 17

    base_w = wid * per_w

    def make_gidx(b):
        for j in range(C // 16):
            v = idx_b[b][pl.ds(16 * j, 16)]
            gidx_b[b][pl.ds(16 * j, 16)] = lax.shift_right_logical(v, 1)
            hb_b[b][pl.ds(16 * j, 16)] = v & 1

    def compute(cur, b):
        row = row_b[b]
        o = out_b[b]
        hb = hb_b[b]
        sb = lax.rem(base_w + cur * C, seq)

        def group(t, _):
            tb = t * G
            idxv = hb[pl.ds(tb, 16)]
            for r in range(G):
                tr = tb + r
                sel = jnp.full((16,), r, jnp.int32)
                hmask = _lane_shuffle(idxv, sel) > 0
                sr = sb + tr
                sr = jnp.where(sr >= seq, sr - seq, sr)
                xs = [jnp.where(hmask,
                                row[tr, pl.ds(D + 16 * k, 16)],
                                row[tr, pl.ds(16 * k, 16)])
                      + pos_v[sr, pl.ds(16 * k, 16)] for k in range(4)]
                for k in range(4):
                    o[tr, pl.ds(16 * k, 16)] = xs[k]
                sv = (xs[0] + xs[1]) + (xs[2] + xs[3])
                qv = (xs[0] * xs[0] + xs[1] * xs[1]) + \
                     (xs[2] * xs[2] + xs[3] * xs[3])
                plsc.store_scatter(s_v, [iota + r * 17], sv)
                plsc.store_scatter(q_v, [iota + r * 17], qv)
            sacc = jnp.zeros((16,), jnp.float32)
            qacc = jnp.zeros((16,), jnp.float32)
            for l in range(16):
                fl = iota17 + l
                sacc = sacc + plsc.load_gather(s_v, [fl])
                qacc = qacc + plsc.load_gather(q_v, [fl])
            mean = sacc * (1.0 / D)
            var = qacc * (1.0 / D) - mean * mean
            rstd = _rsqrt(var + EPS)
            for r in range(G):
                tr = tb + r
                sel = jnp.full((16,), r, jnp.int32)
                mv = _lane_shuffle(mean, sel)
                rv = _lane_shuffle(rstd, sel)
                for k in range(4):
                    x = o[tr, pl.ds(16 * k, 16)]
                    o[tr, pl.ds(16 * k, 16)] = (x - mv) * rv * gs[k] + bs[k]
            return 0

        lax.fori_loop(0, C // G, group, 0)

    def ids_at(c):
        return ids.at[pl.ds(base_w + c * C, C)]

    def out_at(c):
        return out.at[pl.ds(base_w + c * C, C)]

    def gather(b):
        return pltpu.async_copy(tok2.at[gidx_b[b]], row_b[b], gsem[b])

    pltpu.sync_copy(ids_at(0), idx_b[0])
    make_gidx(0)
    gather(0)
    pltpu.async_copy(ids_at(1), idx_b[1], isem[1])

    def step(i, _):
        g2 = i * 2
        for b in (0, 1):
            cur = g2 + b
            ob = 1 - b
            nxt = cur + 1
            @pl.when(nxt < nch)
            def _():
                pltpu.make_async_copy(ids_at(nxt), idx_b[ob], isem[ob]).wait()
                make_gidx(ob)
                gather(ob)
            pltpu.make_async_copy(tok2.at[gidx_b[b]], row_b[b], gsem[b]).wait()
            @pl.when(nxt + 1 < nch)
            def _():
                pltpu.async_copy(ids_at(nxt + 1), idx_b[b], isem[b])
            @pl.when(cur >= 2)
            def _():
                pltpu.make_async_copy(out_b[b], out_at(0), wsem[b]).wait()
            compute(cur, b)
            pltpu.async_copy(out_b[b], out_at(cur), wsem[b])
        return 0

    lax.fori_loop(0, nch // 2, step, 0)
    pltpu.make_async_copy(out_b[0], out_at(0), wsem[0]).wait()
    pltpu.make_async_copy(out_b[1], out_at(0), wsem[1]).wait()


@functools.lru_cache(maxsize=None)
def _build(nrows, seq):
    assert nrows % (NW * C) == 0 and (nrows // (NW * C)) % 2 == 0
    nch = nrows // (NW * C)
    mesh = plsc.VectorSubcoreMesh(
        core_axis_name="c", subcore_axis_name="s",
        num_cores=NC, num_subcores=NS)

    def body(ids, tok2, pos, gam, bet, out,
             pos_v, g_v, b_v, idx0, idx1, gidx0, gidx1, hb0, hb1,
             row0, row1, out0, out1, s_v, q_v,
             gsem0, gsem1, isem0, isem1, wsem0, wsem1):
        _body(seq, nch, ids, tok2, pos, gam, bet, out,
              pos_v, g_v, b_v, (idx0, idx1), (gidx0, gidx1), (hb0, hb1),
              (row0, row1), (out0, out1),
              s_v, q_v, (gsem0, gsem1), (isem0, isem1), (wsem0, wsem1))

    return pl.kernel(
        body,
        out_type=jax.ShapeDtypeStruct((nrows, D), jnp.float32),
        mesh=mesh,
        compiler_params=pltpu.CompilerParams(needs_layout_passes=False),
        scratch_types=[
            pltpu.VMEM((seq, D), jnp.float32),
            pltpu.VMEM((D,), jnp.float32),
            pltpu.VMEM((D,), jnp.float32),
            pltpu.VMEM((C,), jnp.int32),
            pltpu.VMEM((C,), jnp.int32),
            pltpu.VMEM((C,), jnp.int32),
            pltpu.VMEM((C,), jnp.int32),
            pltpu.VMEM((C,), jnp.int32),
            pltpu.VMEM((C,), jnp.int32),
            pltpu.VMEM((C, 2 * D), jnp.float32),
            pltpu.VMEM((C, 2 * D), jnp.float32),
            pltpu.VMEM((C, D), jnp.float32),
            pltpu.VMEM((C, D), jnp.float32),
            pltpu.VMEM((G * 17,), jnp.float32),
            pltpu.VMEM((G * 17,), jnp.float32),
            pltpu.SemaphoreType.DMA,
            pltpu.SemaphoreType.DMA,
            pltpu.SemaphoreType.DMA,
            pltpu.SemaphoreType.DMA,
            pltpu.SemaphoreType.DMA,
            pltpu.SemaphoreType.DMA,
        ],
    )


def kernel(input_ids, token_table, pos_table, gamma, beta):
    batch, seq = input_ids.shape
    vocab = token_table.shape[0]
    ids_flat = input_ids.reshape(-1).astype(jnp.int32)
    tok2 = token_table.reshape(vocab // 2, 2 * D)
    out_flat = _build(batch * seq, seq)(
        ids_flat, tok2, pos_table, gamma, beta)
    return out_flat.reshape(batch, seq, D)
